# Initial kernel scaffold; baseline (speedup 1.0000x reference)
import jax
import jax.numpy as jnp
from jax.experimental import pallas as pl


def kernel(x, w1_p, b1_r, w2, b2_r, w3, b3_r):
    raise NotImplementedError("write your pallas kernel here")


# speedup vs baseline: 1.2420x; 1.2420x over previous
import functools

import jax
import jax.numpy as jnp
from jax.experimental import pallas as pl
from jax.experimental.pallas import tpu as pltpu

_NEG_SLOPE = 0.01
_TK = 4096


def _lrelu(v):
    return jnp.where(v >= 0, v, _NEG_SLOPE * v)


def _mlp_kernel(x_ref, w1_ref, b1_ref, w2_ref, b2_ref, w3_ref, b3_ref,
                o_ref, acc_ref, *, n_k, k_true):
    k = pl.program_id(1)

    @pl.when(k == 0)
    def _init():
        acc_ref[...] = jnp.zeros_like(acc_ref)

    @pl.when(k < n_k - 1)
    def _body():
        acc_ref[...] += jnp.dot(x_ref[...], w1_ref[...],
                                preferred_element_type=jnp.float32)

    @pl.when(k == n_k - 1)
    def _tail():
        tk = x_ref.shape[1]
        cols = jax.lax.broadcasted_iota(jnp.int32, x_ref.shape, 1)
        xm = jnp.where(cols < k_true - (n_k - 1) * tk, x_ref[...], 0.0)
        h1 = acc_ref[...] + jnp.dot(xm, w1_ref[...],
                                    preferred_element_type=jnp.float32)
        h1 = _lrelu(h1 + b1_ref[...])
        h2 = _lrelu(jnp.dot(h1, w2_ref[...],
                            preferred_element_type=jnp.float32)
                    + b2_ref[...])
        o_ref[...] = (jnp.dot(h2, w3_ref[...],
                              preferred_element_type=jnp.float32)
                      + b3_ref[...]).astype(o_ref.dtype)


@jax.jit
def kernel(x, w1_p, b1_r, w2, b2_r, w3, b3_r):
    B, K = x.shape
    N1 = w1_p.shape[1]
    n_split = 2
    bm = B // n_split
    n_k = pl.cdiv(K, _TK)
    assert w1_p.shape[0] >= n_k * _TK

    body = functools.partial(_mlp_kernel, n_k=n_k, k_true=K)
    return pl.pallas_call(
        body,
        out_shape=jax.ShapeDtypeStruct((B, w3.shape[1]), x.dtype),
        grid=(n_split, n_k),
        in_specs=[
            pl.BlockSpec((bm, _TK), lambda i, k: (i, k)),
            pl.BlockSpec((_TK, N1), lambda i, k: (k, 0)),
            pl.BlockSpec(b1_r.shape, lambda i, k: (0, 0)),
            pl.BlockSpec(w2.shape, lambda i, k: (0, 0)),
            pl.BlockSpec(b2_r.shape, lambda i, k: (0, 0)),
            pl.BlockSpec(w3.shape, lambda i, k: (0, 0)),
            pl.BlockSpec(b3_r.shape, lambda i, k: (0, 0)),
        ],
        out_specs=pl.BlockSpec((bm, w3.shape[1]), lambda i, k: (i, 0)),
        scratch_shapes=[pltpu.VMEM((bm, N1), jnp.float32)],
        compiler_params=pltpu.CompilerParams(
            dimension_semantics=("parallel", "arbitrary"),
            vmem_limit_bytes=64 * 1024 * 1024),
    )(x, w1_p, b1_r, w2, b2_r, w3, b3_r)


# speedup vs baseline: 3.7235x; 2.9979x over previous
import functools

import jax
import jax.numpy as jnp
from jax.experimental import pallas as pl
from jax.experimental.pallas import tpu as pltpu

_NEG_SLOPE = 0.01
_TK = 4096


def _lrelu(v):
    return jnp.where(v >= 0, v, _NEG_SLOPE * v)


def _mlp_t_kernel(xt_ref, w1t_ref, b1_ref, w2t_ref, b2_ref, w3t_ref, b3_ref,
                  ot_ref, acc_ref, *, n_k, k_true):
    k = pl.program_id(1)

    @pl.when(k == 0)
    def _init():
        acc_ref[...] = jnp.zeros_like(acc_ref)

    def _fc1_dot(xt_blk):
        return jax.lax.dot_general(
            w1t_ref[...], xt_blk, (((1,), (0,)), ((), ())),
            preferred_element_type=jnp.float32)

    @pl.when(k < n_k - 1)
    def _body():
        acc_ref[...] += _fc1_dot(xt_ref[...])

    @pl.when(k == n_k - 1)
    def _tail():
        tk = xt_ref.shape[0]
        rows = jax.lax.broadcasted_iota(jnp.int32, xt_ref.shape, 0)
        xm = jnp.where(rows < k_true - (n_k - 1) * tk, xt_ref[...], 0.0)
        h1 = _lrelu(acc_ref[...] + _fc1_dot(xm) + b1_ref[...])
        h2 = _lrelu(jnp.dot(w2t_ref[...], h1,
                            preferred_element_type=jnp.float32)
                    + b2_ref[...])
        ot_ref[...] = (jnp.dot(w3t_ref[...], h2,
                               preferred_element_type=jnp.float32)
                       + b3_ref[...]).astype(ot_ref.dtype)


@jax.jit
def kernel(x, w1_p, b1_r, w2, b2_r, w3, b3_r):
    B, K = x.shape
    N1 = w1_p.shape[1]
    n_split = 2
    bn = B // n_split
    n_k = pl.cdiv(K, _TK)
    assert w1_p.shape[0] >= n_k * _TK

    xt = x.T
    w1t = w1_p.T
    w2t, w3t = w2.T, w3.T
    b1c, b2c, b3c = b1_r.T, b2_r.T, b3_r.T

    body = functools.partial(_mlp_t_kernel, n_k=n_k, k_true=K)
    ot = pl.pallas_call(
        body,
        out_shape=jax.ShapeDtypeStruct((w3t.shape[0], B), x.dtype),
        grid=(n_split, n_k),
        in_specs=[
            pl.BlockSpec((_TK, bn), lambda i, k: (k, i)),
            pl.BlockSpec((w1t.shape[0], _TK), lambda i, k: (0, k)),
            pl.BlockSpec(b1c.shape, lambda i, k: (0, 0)),
            pl.BlockSpec(w2t.shape, lambda i, k: (0, 0)),
            pl.BlockSpec(b2c.shape, lambda i, k: (0, 0)),
            pl.BlockSpec(w3t.shape, lambda i, k: (0, 0)),
            pl.BlockSpec(b3c.shape, lambda i, k: (0, 0)),
        ],
        out_specs=pl.BlockSpec((w3t.shape[0], bn), lambda i, k: (0, i)),
        scratch_shapes=[pltpu.VMEM((w1t.shape[0], bn), jnp.float32)],
        compiler_params=pltpu.CompilerParams(
            dimension_semantics=("parallel", "arbitrary"),
            vmem_limit_bytes=64 * 1024 * 1024),
    )(xt, w1t, b1c, w2t, b2c, w3t, b3c)
    return ot.T


# speedup vs baseline: 4.3922x; 1.1796x over previous
import functools

import jax
import jax.numpy as jnp
from jax.experimental import pallas as pl
from jax.experimental.pallas import tpu as pltpu

_NEG_SLOPE = 0.01
_TK = 2048


def _lrelu(v):
    return jnp.where(v >= 0, v, _NEG_SLOPE * v)


def _fc1_t_kernel(xt_ref, w1t_ref, p_ref, *, n_k, n_tiles, k_true):
    i, k = pl.program_id(0), pl.program_id(1)
    g = i * n_k + k

    @pl.when(k == 0)
    def _init():
        p_ref[...] = jnp.zeros_like(p_ref)

    def _acc(xt_blk):
        p_ref[0] += jax.lax.dot_general(
            w1t_ref[...], xt_blk, (((1,), (0,)), ((), ())),
            preferred_element_type=jnp.float32)

    @pl.when(g < n_tiles - 1)
    def _body():
        _acc(xt_ref[...])

    @pl.when(g == n_tiles - 1)
    def _tail():
        tk = xt_ref.shape[0]
        rows = jax.lax.broadcasted_iota(jnp.int32, xt_ref.shape, 0)
        _acc(jnp.where(rows < k_true - (n_tiles - 1) * tk, xt_ref[...], 0.0))


def _head_t_kernel(p_ref, b1_ref, w2t_ref, b2_ref, w3t_ref, b3_ref, ot_ref):
    h1 = _lrelu(p_ref[0] + p_ref[1]
                + b1_ref[...].reshape(-1, 1))
    h2 = _lrelu(jnp.dot(w2t_ref[...], h1,
                        preferred_element_type=jnp.float32)
                + b2_ref[...].reshape(-1, 1))
    ot_ref[...] = (jnp.dot(w3t_ref[...], h2,
                           preferred_element_type=jnp.float32)
                   + b3_ref[...].reshape(-1, 1)).astype(ot_ref.dtype)


@jax.jit
def kernel(x, w1_p, b1_r, w2, b2_r, w3, b3_r):
    B, K = x.shape
    N1 = w1_p.shape[1]
    n_split = 2
    n_tiles = pl.cdiv(K, _TK)
    n_k = pl.cdiv(n_tiles, n_split)
    assert w1_p.shape[0] >= n_k * n_split * _TK
    assert n_tiles == n_k * n_split

    xt = x.T
    w1t = w1_p.T
    w2t, w3t = w2.T, w3.T

    fc1 = functools.partial(_fc1_t_kernel, n_k=n_k, n_tiles=n_tiles, k_true=K)
    partial = pl.pallas_call(
        fc1,
        out_shape=jax.ShapeDtypeStruct((n_split, N1, B), jnp.float32),
        grid=(n_split, n_k),
        in_specs=[
            pl.BlockSpec((_TK, B), lambda i, k: (i * (n_tiles // 2) + k, 0)),
            pl.BlockSpec((N1, _TK), lambda i, k: (0, i * (n_tiles // 2) + k)),
        ],
        out_specs=pl.BlockSpec((1, N1, B), lambda i, k: (i, 0, 0)),
        compiler_params=pltpu.CompilerParams(
            dimension_semantics=("parallel", "arbitrary"),
            vmem_limit_bytes=64 * 1024 * 1024),
    )(xt, w1t)

    ot = pl.pallas_call(
        _head_t_kernel,
        out_shape=jax.ShapeDtypeStruct((w3t.shape[0], B), x.dtype),
        grid=(1,),
        in_specs=[
            pl.BlockSpec((n_split, N1, B), lambda i: (0, 0, 0)),
            pl.BlockSpec(b1_r.shape, lambda i: (0, 0)),
            pl.BlockSpec(w2t.shape, lambda i: (0, 0)),
            pl.BlockSpec(b2_r.shape, lambda i: (0, 0)),
            pl.BlockSpec(w3t.shape, lambda i: (0, 0)),
            pl.BlockSpec(b3_r.shape, lambda i: (0, 0)),
        ],
        out_specs=pl.BlockSpec((w3t.shape[0], B), lambda i: (0, 0)),
    )(partial, b1_r, w2t, b2_r, w3t, b3_r)
    return ot.T


# speedup vs baseline: 5.8114x; 1.3231x over previous
import functools

import jax
import jax.numpy as jnp
from jax.experimental import pallas as pl
from jax.experimental.pallas import tpu as pltpu

_NEG_SLOPE = 0.01
_TK = 5632


def _lrelu(v):
    return jnp.where(v >= 0, v, _NEG_SLOPE * v)


def _fc1_t_kernel(xt_ref, w1t_ref, p_ref, *, n_k, n_tiles, k_true):
    i, k = pl.program_id(0), pl.program_id(1)
    g = i * n_k + k

    @pl.when(k == 0)
    def _init():
        p_ref[...] = jnp.zeros_like(p_ref)

    def _acc(xt_blk):
        p_ref[0] += jax.lax.dot_general(
            w1t_ref[...], xt_blk, (((1,), (0,)), ((), ())),
            preferred_element_type=jnp.float32)

    @pl.when(g < n_tiles - 1)
    def _body():
        _acc(xt_ref[...])

    @pl.when(g == n_tiles - 1)
    def _tail():
        tk = xt_ref.shape[0]
        rows = jax.lax.broadcasted_iota(jnp.int32, xt_ref.shape, 0)
        _acc(jnp.where(rows < k_true - (n_tiles - 1) * tk, xt_ref[...], 0.0))


def _head_t_kernel(p_ref, b1_ref, w2t_ref, b2_ref, w3t_ref, b3_ref, ot_ref):
    h1 = _lrelu(p_ref[0] + p_ref[1]
                + b1_ref[...].reshape(-1, 1))
    h2 = _lrelu(jnp.dot(w2t_ref[...], h1,
                        preferred_element_type=jnp.float32)
                + b2_ref[...].reshape(-1, 1))
    ot_ref[...] = (jnp.dot(w3t_ref[...], h2,
                           preferred_element_type=jnp.float32)
                   + b3_ref[...].reshape(-1, 1)).astype(ot_ref.dtype)


@jax.jit
def kernel(x, w1_p, b1_r, w2, b2_r, w3, b3_r):
    B, K = x.shape
    N1 = w1_p.shape[1]
    n_split = 2
    n_tiles = pl.cdiv(K, _TK)
    n_k = pl.cdiv(n_tiles, n_split)
    assert w1_p.shape[0] >= n_k * n_split * _TK
    assert n_tiles == n_k * n_split

    xt = x.T
    w1t = w1_p.T
    w2t, w3t = w2.T, w3.T

    fc1 = functools.partial(_fc1_t_kernel, n_k=n_k, n_tiles=n_tiles, k_true=K)
    partial = pl.pallas_call(
        fc1,
        out_shape=jax.ShapeDtypeStruct((n_split, N1, B), jnp.float32),
        grid=(n_split, n_k),
        in_specs=[
            pl.BlockSpec((_TK, B), lambda i, k: (i * (n_tiles // 2) + k, 0)),
            pl.BlockSpec((N1, _TK), lambda i, k: (0, i * (n_tiles // 2) + k)),
        ],
        out_specs=pl.BlockSpec((1, N1, B), lambda i, k: (i, 0, 0)),
        compiler_params=pltpu.CompilerParams(
            dimension_semantics=("parallel", "arbitrary"),
            vmem_limit_bytes=64 * 1024 * 1024),
    )(xt, w1t)

    ot = pl.pallas_call(
        _head_t_kernel,
        out_shape=jax.ShapeDtypeStruct((w3t.shape[0], B), x.dtype),
        grid=(1,),
        in_specs=[
            pl.BlockSpec((n_split, N1, B), lambda i: (0, 0, 0)),
            pl.BlockSpec(b1_r.shape, lambda i: (0, 0)),
            pl.BlockSpec(w2t.shape, lambda i: (0, 0)),
            pl.BlockSpec(b2_r.shape, lambda i: (0, 0)),
            pl.BlockSpec(w3t.shape, lambda i: (0, 0)),
            pl.BlockSpec(b3_r.shape, lambda i: (0, 0)),
        ],
        out_specs=pl.BlockSpec((w3t.shape[0], B), lambda i: (0, 0)),
    )(partial, b1_r, w2t, b2_r, w3t, b3_r)
    return ot.T
